# baseline (device time: 161983 ns/iter reference)
import jax
import jax.numpy as jnp
from jax import lax
from jax.experimental import pallas as pl
from jax.experimental.pallas import tpu as pltpu

N_DEV = 4
V_PER = 16384
N_IDX = 2048
D = 1024
HALF = N_IDX // 2
CHUNK = HALF // N_DEV
GATHER_SEMS = 16


def kernel(table, idx):
    my_pos = lax.axis_index("i")
    off = (my_pos * V_PER).astype(jnp.int32)
    local_idx = jnp.clip(idx - off, 0, V_PER - 1).astype(jnp.int32)
    mask2d = ((idx >= off) & (idx < off + V_PER)).astype(jnp.float32)
    mask2d = mask2d.reshape(N_IDX, 1)

    def body(table_ref, lidx_ref, mask_ref, out_ref,
             rs_buf_r, rs_buf_l,
             r_send_sems, r_recv_sems, l_send_sems, l_recv_sems,
             gather_sems):
        p = lax.axis_index("i")
        left = lax.rem(p - 1 + N_DEV, N_DEV)
        right = lax.rem(p + 1, N_DEV)

        def gwait(k):
            pltpu.make_async_copy(
                table_ref.at[pl.ds(0, 1), :],
                out_ref.at[pl.ds(0, 1), :],
                gather_sems.at[k],
            ).wait()

        def gather_region(base):
            def blk(b, carry):
                for k in range(GATHER_SEMS):
                    @pl.when(b > 0)
                    def _():
                        gwait(k)

                    pos = base + b * GATHER_SEMS + k
                    pltpu.make_async_copy(
                        table_ref.at[pl.ds(lidx_ref[pos], 1), :],
                        out_ref.at[pl.ds(pos, 1), :],
                        gather_sems.at[k],
                    ).start()
                return carry

            lax.fori_loop(0, CHUNK // GATHER_SEMS, blk, 0)
            for k in range(GATHER_SEMS):
                gwait(k)
            out_ref[pl.ds(base, CHUNK), :] = (
                out_ref[pl.ds(base, CHUNK), :] * mask_ref[pl.ds(base, CHUNK), :]
            )

        def r_base(c):
            return c * CHUNK

        def l_base(c):
            return HALF + c * CHUNK

        barrier_sem = pltpu.get_barrier_semaphore()
        for nbr in (left, right):
            pl.semaphore_signal(
                barrier_sem, inc=1,
                device_id=(nbr,), device_id_type=pl.DeviceIdType.MESH,
            )
        pl.semaphore_wait(barrier_sem, 2)

        gather_region(r_base(p))
        gather_region(l_base(p))

        for s in range(N_DEV - 1):
            sc_r = lax.rem(p - s + N_DEV, N_DEV)
            rc_r = lax.rem(p - s - 1 + N_DEV, N_DEV)
            sc_l = lax.rem(p + s, N_DEV)
            rc_l = lax.rem(p + s + 1, N_DEV)
            rdma_r = pltpu.make_async_remote_copy(
                src_ref=out_ref.at[pl.ds(r_base(sc_r), CHUNK), :],
                dst_ref=rs_buf_r.at[s],
                send_sem=r_send_sems.at[s],
                recv_sem=r_recv_sems.at[s],
                device_id=(right,),
                device_id_type=pl.DeviceIdType.MESH,
            )
            rdma_l = pltpu.make_async_remote_copy(
                src_ref=out_ref.at[pl.ds(l_base(sc_l), CHUNK), :],
                dst_ref=rs_buf_l.at[s],
                send_sem=l_send_sems.at[s],
                recv_sem=l_recv_sems.at[s],
                device_id=(left,),
                device_id_type=pl.DeviceIdType.MESH,
            )
            rdma_r.start()
            rdma_l.start()
            gather_region(r_base(rc_r))
            gather_region(l_base(rc_l))
            rdma_r.wait_recv()
            out_ref[pl.ds(r_base(rc_r), CHUNK), :] = (
                out_ref[pl.ds(r_base(rc_r), CHUNK), :] + rs_buf_r[s]
            )
            rdma_l.wait_recv()
            out_ref[pl.ds(l_base(rc_l), CHUNK), :] = (
                out_ref[pl.ds(l_base(rc_l), CHUNK), :] + rs_buf_l[s]
            )
            rdma_r.wait_send()
            rdma_l.wait_send()

        for s in range(N_DEV - 1):
            j = (N_DEV - 1) + s
            sc_r = lax.rem(p + 1 - s + N_DEV, N_DEV)
            rc_r = lax.rem(p - s + N_DEV, N_DEV)
            sc_l = lax.rem(p - 1 + s + N_DEV, N_DEV)
            rc_l = lax.rem(p + s, N_DEV)
            send_r = pltpu.make_async_remote_copy(
                src_ref=out_ref.at[pl.ds(r_base(sc_r), CHUNK), :],
                dst_ref=out_ref.at[pl.ds(r_base(sc_r), CHUNK), :],
                send_sem=r_send_sems.at[j],
                recv_sem=r_recv_sems.at[j],
                device_id=(right,),
                device_id_type=pl.DeviceIdType.MESH,
            )
            send_l = pltpu.make_async_remote_copy(
                src_ref=out_ref.at[pl.ds(l_base(sc_l), CHUNK), :],
                dst_ref=out_ref.at[pl.ds(l_base(sc_l), CHUNK), :],
                send_sem=l_send_sems.at[j],
                recv_sem=l_recv_sems.at[j],
                device_id=(left,),
                device_id_type=pl.DeviceIdType.MESH,
            )
            send_r.start()
            send_l.start()
            recv_r = pltpu.make_async_remote_copy(
                src_ref=out_ref.at[pl.ds(r_base(sc_r), CHUNK), :],
                dst_ref=out_ref.at[pl.ds(r_base(rc_r), CHUNK), :],
                send_sem=r_send_sems.at[j],
                recv_sem=r_recv_sems.at[j],
                device_id=(left,),
                device_id_type=pl.DeviceIdType.MESH,
            )
            recv_l = pltpu.make_async_remote_copy(
                src_ref=out_ref.at[pl.ds(l_base(sc_l), CHUNK), :],
                dst_ref=out_ref.at[pl.ds(l_base(rc_l), CHUNK), :],
                send_sem=l_send_sems.at[j],
                recv_sem=l_recv_sems.at[j],
                device_id=(right,),
                device_id_type=pl.DeviceIdType.MESH,
            )
            recv_r.wait_recv()
            recv_l.wait_recv()
            send_r.wait_send()
            send_l.wait_send()

    return pl.pallas_call(
        body,
        out_shape=jax.ShapeDtypeStruct((N_IDX, D), jnp.float32),
        in_specs=[
            pl.BlockSpec(memory_space=pl.ANY),
            pl.BlockSpec(memory_space=pltpu.SMEM),
            pl.BlockSpec(memory_space=pltpu.VMEM),
        ],
        out_specs=pl.BlockSpec(memory_space=pltpu.VMEM),
        scratch_shapes=[
            pltpu.VMEM((N_DEV - 1, CHUNK, D), jnp.float32),
            pltpu.VMEM((N_DEV - 1, CHUNK, D), jnp.float32),
            pltpu.SemaphoreType.DMA((2 * (N_DEV - 1),)),
            pltpu.SemaphoreType.DMA((2 * (N_DEV - 1),)),
            pltpu.SemaphoreType.DMA((2 * (N_DEV - 1),)),
            pltpu.SemaphoreType.DMA((2 * (N_DEV - 1),)),
            pltpu.SemaphoreType.DMA((GATHER_SEMS,)),
        ],
        compiler_params=pltpu.CompilerParams(collective_id=0),
    )(table, local_idx, mask2d)


# device time: 114763 ns/iter; 1.4115x vs baseline; 1.4115x over previous
import jax
import jax.numpy as jnp
from jax import lax
from jax.experimental import pallas as pl
from jax.experimental.pallas import tpu as pltpu

N_DEV = 4
V_PER = 16384
N_IDX = 2048
D = 1024
HALF = N_IDX // 2
CHUNK = HALF // N_DEV
GATHER_SEMS = 16


def kernel(table, idx):
    my_pos = lax.axis_index("i")
    off = (my_pos * V_PER).astype(jnp.int32)
    local_idx = jnp.clip(idx - off, 0, V_PER - 1).astype(jnp.int32)
    mask2d = ((idx >= off) & (idx < off + V_PER)).astype(jnp.float32)
    mask2d = mask2d.reshape(N_IDX, 1)

    def body(table_ref, lidx_ref, mask_ref, out_ref,
             rs_buf_r, rs_buf_l,
             r_send_sems, r_recv_sems, l_send_sems, l_recv_sems,
             gather_sem):
        p = lax.axis_index("i")
        left = lax.rem(p - 1 + N_DEV, N_DEV)
        right = lax.rem(p + 1, N_DEV)

        def gather_region(base):
            def blk(b, carry):
                for k in range(GATHER_SEMS):
                    pos = base + b * GATHER_SEMS + k
                    pltpu.make_async_copy(
                        table_ref.at[pl.ds(lidx_ref[pos], 1), :],
                        out_ref.at[pl.ds(pos, 1), :],
                        gather_sem,
                    ).start()
                return carry

            lax.fori_loop(0, CHUNK // GATHER_SEMS, blk, 0)
            pltpu.make_async_copy(
                table_ref.at[pl.ds(0, CHUNK), :],
                out_ref.at[pl.ds(base, CHUNK), :],
                gather_sem,
            ).wait()
            out_ref[pl.ds(base, CHUNK), :] = (
                out_ref[pl.ds(base, CHUNK), :] * mask_ref[pl.ds(base, CHUNK), :]
            )

        def r_base(c):
            return c * CHUNK

        def l_base(c):
            return HALF + c * CHUNK

        gather_region(r_base(p))
        gather_region(l_base(p))

        barrier_sem = pltpu.get_barrier_semaphore()
        for nbr in (left, right):
            pl.semaphore_signal(
                barrier_sem, inc=1,
                device_id=(nbr,), device_id_type=pl.DeviceIdType.MESH,
            )
        pl.semaphore_wait(barrier_sem, 2)

        for s in range(N_DEV - 1):
            sc_r = lax.rem(p - s + N_DEV, N_DEV)
            rc_r = lax.rem(p - s - 1 + N_DEV, N_DEV)
            sc_l = lax.rem(p + s, N_DEV)
            rc_l = lax.rem(p + s + 1, N_DEV)
            rdma_r = pltpu.make_async_remote_copy(
                src_ref=out_ref.at[pl.ds(r_base(sc_r), CHUNK), :],
                dst_ref=rs_buf_r.at[s],
                send_sem=r_send_sems.at[s],
                recv_sem=r_recv_sems.at[s],
                device_id=(right,),
                device_id_type=pl.DeviceIdType.MESH,
            )
            rdma_l = pltpu.make_async_remote_copy(
                src_ref=out_ref.at[pl.ds(l_base(sc_l), CHUNK), :],
                dst_ref=rs_buf_l.at[s],
                send_sem=l_send_sems.at[s],
                recv_sem=l_recv_sems.at[s],
                device_id=(left,),
                device_id_type=pl.DeviceIdType.MESH,
            )
            rdma_r.start()
            rdma_l.start()
            gather_region(r_base(rc_r))
            gather_region(l_base(rc_l))
            rdma_r.wait_recv()
            out_ref[pl.ds(r_base(rc_r), CHUNK), :] = (
                out_ref[pl.ds(r_base(rc_r), CHUNK), :] + rs_buf_r[s]
            )
            rdma_l.wait_recv()
            out_ref[pl.ds(l_base(rc_l), CHUNK), :] = (
                out_ref[pl.ds(l_base(rc_l), CHUNK), :] + rs_buf_l[s]
            )
            rdma_r.wait_send()
            rdma_l.wait_send()

        for s in range(N_DEV - 1):
            j = (N_DEV - 1) + s
            sc_r = lax.rem(p + 1 - s + N_DEV, N_DEV)
            rc_r = lax.rem(p - s + N_DEV, N_DEV)
            sc_l = lax.rem(p - 1 + s + N_DEV, N_DEV)
            rc_l = lax.rem(p + s, N_DEV)
            send_r = pltpu.make_async_remote_copy(
                src_ref=out_ref.at[pl.ds(r_base(sc_r), CHUNK), :],
                dst_ref=out_ref.at[pl.ds(r_base(sc_r), CHUNK), :],
                send_sem=r_send_sems.at[j],
                recv_sem=r_recv_sems.at[j],
                device_id=(right,),
                device_id_type=pl.DeviceIdType.MESH,
            )
            send_l = pltpu.make_async_remote_copy(
                src_ref=out_ref.at[pl.ds(l_base(sc_l), CHUNK), :],
                dst_ref=out_ref.at[pl.ds(l_base(sc_l), CHUNK), :],
                send_sem=l_send_sems.at[j],
                recv_sem=l_recv_sems.at[j],
                device_id=(left,),
                device_id_type=pl.DeviceIdType.MESH,
            )
            send_r.start()
            send_l.start()
            recv_r = pltpu.make_async_remote_copy(
                src_ref=out_ref.at[pl.ds(r_base(sc_r), CHUNK), :],
                dst_ref=out_ref.at[pl.ds(r_base(rc_r), CHUNK), :],
                send_sem=r_send_sems.at[j],
                recv_sem=r_recv_sems.at[j],
                device_id=(left,),
                device_id_type=pl.DeviceIdType.MESH,
            )
            recv_l = pltpu.make_async_remote_copy(
                src_ref=out_ref.at[pl.ds(l_base(sc_l), CHUNK), :],
                dst_ref=out_ref.at[pl.ds(l_base(rc_l), CHUNK), :],
                send_sem=l_send_sems.at[j],
                recv_sem=l_recv_sems.at[j],
                device_id=(right,),
                device_id_type=pl.DeviceIdType.MESH,
            )
            recv_r.wait_recv()
            recv_l.wait_recv()
            send_r.wait_send()
            send_l.wait_send()

    return pl.pallas_call(
        body,
        out_shape=jax.ShapeDtypeStruct((N_IDX, D), jnp.float32),
        in_specs=[
            pl.BlockSpec(memory_space=pl.ANY),
            pl.BlockSpec(memory_space=pltpu.SMEM),
            pl.BlockSpec(memory_space=pltpu.VMEM),
        ],
        out_specs=pl.BlockSpec(memory_space=pltpu.VMEM),
        scratch_shapes=[
            pltpu.VMEM((N_DEV - 1, CHUNK, D), jnp.float32),
            pltpu.VMEM((N_DEV - 1, CHUNK, D), jnp.float32),
            pltpu.SemaphoreType.DMA((2 * (N_DEV - 1),)),
            pltpu.SemaphoreType.DMA((2 * (N_DEV - 1),)),
            pltpu.SemaphoreType.DMA((2 * (N_DEV - 1),)),
            pltpu.SemaphoreType.DMA((2 * (N_DEV - 1),)),
            pltpu.SemaphoreType.DMA,
        ],
        compiler_params=pltpu.CompilerParams(collective_id=0),
    )(table, local_idx, mask2d)


# device time: 109066 ns/iter; 1.4852x vs baseline; 1.0522x over previous
import jax
import jax.numpy as jnp
from jax import lax
from jax.experimental import pallas as pl
from jax.experimental.pallas import tpu as pltpu

N_DEV = 4
V_PER = 16384
N_IDX = 2048
D = 1024
HALF = N_IDX // 2
CHUNK = HALF // N_DEV
SUB = CHUNK // 2
GATHER_SEMS = 16
MESH = pl.DeviceIdType.MESH


def kernel(table, idx):
    my_pos = lax.axis_index("i")
    off = (my_pos * V_PER).astype(jnp.int32)
    local_idx = jnp.clip(idx - off, 0, V_PER - 1).astype(jnp.int32)
    mask2d = ((idx >= off) & (idx < off + V_PER)).astype(jnp.float32)
    mask2d = mask2d.reshape(N_IDX, 1)

    def body(table_ref, lidx_ref, mask_ref, out_ref,
             rs_buf_r, rs_buf_l,
             r_send_sems, r_recv_sems, l_send_sems, l_recv_sems,
             gather_sem):
        p = lax.axis_index("i")
        left = lax.rem(p - 1 + N_DEV, N_DEV)
        right = lax.rem(p + 1, N_DEV)

        def cmod(x):
            return lax.rem(x + 2 * N_DEV, N_DEV)

        def sub_rows(ring, c, u):
            return pl.ds(ring * HALF + c * CHUNK + u * SUB, SUB)

        def chunk_rows(ring, c):
            return pl.ds(ring * HALF + c * CHUNK, CHUNK)

        def gather_issue(ring, c):
            base = ring * HALF + c * CHUNK

            def blk(b, carry):
                for k in range(GATHER_SEMS):
                    pos = base + b * GATHER_SEMS + k
                    pltpu.make_async_copy(
                        table_ref.at[pl.ds(lidx_ref[pos], 1), :],
                        out_ref.at[pl.ds(pos, 1), :],
                        gather_sem,
                    ).start()
                return carry

            lax.fori_loop(0, CHUNK // GATHER_SEMS, blk, 0)

        def gather_finish(ring, c):
            pltpu.make_async_copy(
                table_ref.at[pl.ds(0, CHUNK), :],
                out_ref.at[chunk_rows(ring, c), :],
                gather_sem,
            ).wait()
            out_ref[chunk_rows(ring, c), :] = (
                out_ref[chunk_rows(ring, c), :]
                * mask_ref[chunk_rows(ring, c), :]
            )

        def gather(ring, c):
            gather_issue(ring, c)
            gather_finish(ring, c)

        def rs_desc(ring, s, u, c):
            return pltpu.make_async_remote_copy(
                src_ref=out_ref.at[sub_rows(ring, c, u), :],
                dst_ref=(rs_buf_r if ring == 0 else rs_buf_l).at[
                    s, pl.ds(u * SUB, SUB), :
                ],
                send_sem=(r_send_sems if ring == 0 else l_send_sems).at[
                    2 * s + u
                ],
                recv_sem=(r_recv_sems if ring == 0 else l_recv_sems).at[
                    2 * s + u
                ],
                device_id=((right,) if ring == 0 else (left,)),
                device_id_type=MESH,
            )

        def ag_desc(ring, h, u, c_src, c_dst):
            return pltpu.make_async_remote_copy(
                src_ref=out_ref.at[sub_rows(ring, c_src, u), :],
                dst_ref=out_ref.at[sub_rows(ring, c_dst, u), :],
                send_sem=(r_send_sems if ring == 0 else l_send_sems).at[
                    6 + 2 * h + u
                ],
                recv_sem=(r_recv_sems if ring == 0 else l_recv_sems).at[
                    6 + 2 * h + u
                ],
                device_id=((right,) if ring == 0 else (left,)),
                device_id_type=MESH,
            )

        def rs_send_chunk(ring, s):
            return cmod(p - s) if ring == 0 else cmod(p + s)

        def rs_recv_chunk(ring, s):
            return cmod(p - s - 1) if ring == 0 else cmod(p + s + 1)

        def ag_send_chunk(ring, h):
            return cmod(p + 1 - h) if ring == 0 else cmod(p - 1 + h)

        def ag_recv_chunk(ring, h):
            return cmod(p - h) if ring == 0 else cmod(p + h)

        gather_issue(0, p)
        gather_issue(1, p)

        barrier_sem = pltpu.get_barrier_semaphore()
        for nbr in (left, right):
            pl.semaphore_signal(
                barrier_sem, inc=1, device_id=(nbr,), device_id_type=MESH,
            )
        pl.semaphore_wait(barrier_sem, 2)

        gather_finish(0, p)
        gather_finish(1, p)

        for ring in (0, 1):
            for u in (0, 1):
                rs_desc(ring, 0, u, rs_send_chunk(ring, 0)).start()
        gather(0, rs_recv_chunk(0, 0))
        gather(1, rs_recv_chunk(1, 0))

        for s in range(N_DEV - 1):
            for u in (0, 1):
                for ring in (0, 1):
                    rc = rs_recv_chunk(ring, s)
                    rs_desc(ring, s, u, rs_send_chunk(ring, s)).wait_recv()
                    buf = rs_buf_r if ring == 0 else rs_buf_l
                    out_ref[sub_rows(ring, rc, u), :] = (
                        out_ref[sub_rows(ring, rc, u), :]
                        + buf[s, pl.ds(u * SUB, SUB), :]
                    )
                    if s < N_DEV - 2:
                        rs_desc(ring, s + 1, u, rc).start()
                    else:
                        ag_desc(ring, 0, u, rc, rc).start()
            if s < N_DEV - 2:
                gather(0, rs_recv_chunk(0, s + 1))
                gather(1, rs_recv_chunk(1, s + 1))

        for h in range(1, N_DEV - 1):
            for u in (0, 1):
                for ring in (0, 1):
                    c_in = ag_recv_chunk(ring, h - 1)
                    ag_desc(
                        ring, h - 1, u, ag_send_chunk(ring, h - 1), c_in
                    ).wait_recv()
                    ag_desc(ring, h, u, c_in, c_in).start()
        for u in (0, 1):
            for ring in (0, 1):
                ag_desc(
                    ring,
                    N_DEV - 2,
                    u,
                    ag_send_chunk(ring, N_DEV - 2),
                    ag_recv_chunk(ring, N_DEV - 2),
                ).wait_recv()

        for ring in (0, 1):
            for s in range(N_DEV - 1):
                for u in (0, 1):
                    rs_desc(ring, s, u, rs_send_chunk(ring, s)).wait_send()
                    ag_desc(
                        ring, s, u, ag_send_chunk(ring, s),
                        ag_send_chunk(ring, s),
                    ).wait_send()

    return pl.pallas_call(
        body,
        out_shape=jax.ShapeDtypeStruct((N_IDX, D), jnp.float32),
        in_specs=[
            pl.BlockSpec(memory_space=pl.ANY),
            pl.BlockSpec(memory_space=pltpu.SMEM),
            pl.BlockSpec(memory_space=pltpu.VMEM),
        ],
        out_specs=pl.BlockSpec(memory_space=pltpu.VMEM),
        scratch_shapes=[
            pltpu.VMEM((N_DEV - 1, CHUNK, D), jnp.float32),
            pltpu.VMEM((N_DEV - 1, CHUNK, D), jnp.float32),
            pltpu.SemaphoreType.DMA((12,)),
            pltpu.SemaphoreType.DMA((12,)),
            pltpu.SemaphoreType.DMA((12,)),
            pltpu.SemaphoreType.DMA((12,)),
            pltpu.SemaphoreType.DMA,
        ],
        compiler_params=pltpu.CompilerParams(collective_id=0),
    )(table, local_idx, mask2d)
